# baseline (speedup 1.0000x reference)
import jax
import jax.numpy as jnp
from jax import lax
from jax.experimental import pallas as pl
from jax.experimental.pallas import tpu as pltpu
from jax.experimental.pallas import tpu_sc as plsc

N_NODES = 10000
HIDDEN = 128
NUM_CLASSES = 4
N_EDGES = 320000

NUM_CORES = 2
NUM_SUBCORES = 16
NUM_TILES = NUM_CORES * NUM_SUBCORES
EDGES_PER_TILE = N_EDGES // NUM_TILES
CHUNK = 80
CHUNKS_PER_TILE = EDGES_PER_TILE // CHUNK
IDX_GROUPS = 5
GCHUNKS = CHUNKS_PER_TILE // IDX_GROUPS
NBUF = 3
ACC_ROWS = N_NODES
ZERO_SUBCORES = 10
ZERO_ROWS = ACC_ROWS // ZERO_SUBCORES

ROW_BLOCK = 400
GRID = N_NODES // ROW_BLOCK
PART_BLOCKS = ACC_ROWS // ROW_BLOCK


def _sc_aggregate(q, src, dst, zeros, d, gather):
  mesh = plsc.VectorSubcoreMesh(core_axis_name="c", subcore_axis_name="s")

  def body(*refs):
    if gather:
      (q_hbm, srcr_hbm, dstr_hbm, z_hbm, out_hbm,
       srcs_v, dsts_v, r0b, r1b, r2b, acc,
       sg0, sg1, sg2, ss0, ss1, ss2) = refs
      rows = (r0b, r1b, r2b)
      semg = (sg0, sg1, sg2)
      sems = (ss0, ss1, ss2)
    else:
      dstr_hbm, z_hbm, out_hbm, dsts_v, r0b, acc, ss0 = refs
      rows = (r0b,)
      sems = (ss0,)
    c = lax.axis_index("c")
    s = lax.axis_index("s")
    wid = c * NUM_SUBCORES + s

    r0 = s * ZERO_ROWS

    @pl.when(s < ZERO_SUBCORES)
    def _zero():
      pltpu.sync_copy(z_hbm.at[pl.ds(r0, ZERO_ROWS)],
                      acc.at[pl.ds(r0, ZERO_ROWS)])

    if not gather:
      @pl.loop(0, CHUNK)
      def _fill(r):
        for j in range(d // 16):
          rows[0][r, pl.ds(j * 16, 16)] = jnp.ones((16,), jnp.float32)

    plsc.subcore_barrier()

    t0 = wid * CHUNKS_PER_TILE

    if gather:
      @pl.loop(0, IDX_GROUPS)
      def _group(g):
        pltpu.sync_copy(dstr_hbm.at[pl.ds(t0 + g * GCHUNKS, GCHUNKS)], dsts_v)
        pltpu.sync_copy(srcr_hbm.at[pl.ds(t0 + g * GCHUNKS, GCHUNKS)], srcs_v)
        pltpu.async_copy(q_hbm.at[srcs_v.at[0]], rows[0], semg[0])
        pltpu.async_copy(q_hbm.at[srcs_v.at[1]], rows[1], semg[1])

        @pl.loop(0, GCHUNKS // NBUF)
        def _ring(tt):
          for b in range(NBUF):
            j = tt * NBUF + b
            p = (b + 2) % NBUF
            pltpu.make_async_copy(q_hbm.at[srcs_v.at[j]], rows[b],
                                  semg[b]).wait()
            pltpu.async_copy(rows[b], acc.at[dsts_v.at[j]], sems[b], add=True)

            @pl.when(jnp.logical_and(j + 2 <= GCHUNKS - 1, j >= 1))
            def _reuse():
              pltpu.make_async_copy(rows[p], acc.at[dsts_v.at[j - 1]],
                                    sems[p]).wait()
              pltpu.async_copy(q_hbm.at[srcs_v.at[j + 2]], rows[p], semg[p])

            @pl.when(jnp.logical_and(j + 2 <= GCHUNKS - 1, j < 1))
            def _first():
              pltpu.async_copy(q_hbm.at[srcs_v.at[j + 2]], rows[p], semg[p])

        last = GCHUNKS - 1
        pltpu.make_async_copy(q_hbm.at[srcs_v.at[last]], rows[0],
                              semg[0]).wait()
        pltpu.async_copy(rows[0], acc.at[dsts_v.at[last]], sems[0], add=True)
        for b in (1, 2, 0):
          j = GCHUNKS - 3 + ((b - 1) % NBUF)
          pltpu.make_async_copy(rows[b], acc.at[dsts_v.at[j]], sems[b]).wait()
    else:
      @pl.loop(0, IDX_GROUPS)
      def _dgroup(g):
        pltpu.sync_copy(dstr_hbm.at[pl.ds(t0 + g * GCHUNKS, GCHUNKS)], dsts_v)

        @pl.loop(0, GCHUNKS, step=5)
        def _deg_batch(j):
          for b in range(5):
            pltpu.async_copy(rows[0], acc.at[dsts_v.at[j + b]], sems[0],
                             add=True)
          for b in range(5):
            pltpu.make_async_copy(rows[0], acc.at[dsts_v.at[j + b]],
                                  sems[0]).wait()

    plsc.subcore_barrier()

    @pl.when(s < ZERO_SUBCORES)
    def _writeback():
      pltpu.sync_copy(acc.at[pl.ds(r0, ZERO_ROWS)],
                      out_hbm.at[pl.ds(c * ACC_ROWS + r0, ZERO_ROWS)])

  if gather:
    scratch = [
        pltpu.VMEM((GCHUNKS, CHUNK), jnp.int32),
        pltpu.VMEM((GCHUNKS, CHUNK), jnp.int32),
        pltpu.VMEM((CHUNK, d), jnp.float32),
        pltpu.VMEM((CHUNK, d), jnp.float32),
        pltpu.VMEM((CHUNK, d), jnp.float32),
        pltpu.VMEM_SHARED((ACC_ROWS, d), jnp.float32),
    ] + [pltpu.SemaphoreType.DMA] * 6
  else:
    scratch = [
        pltpu.VMEM((GCHUNKS, CHUNK), jnp.int32),
        pltpu.VMEM((CHUNK, d), jnp.float32),
        pltpu.VMEM_SHARED((ACC_ROWS, d), jnp.float32),
        pltpu.SemaphoreType.DMA,
    ]
  k = pl.kernel(
      body,
      out_type=jax.ShapeDtypeStruct((NUM_CORES * ACC_ROWS, d), jnp.float32),
      mesh=mesh,
      scratch_types=scratch,
      compiler_params=pltpu.CompilerParams(use_tc_tiling_on_sc=False),
  )
  if gather:
    return k(q, src, dst, zeros)
  return k(dst, zeros)


def _sc_flat_aggregate(q4, srcf, dstf, zeros, gather):
  mesh = plsc.VectorSubcoreMesh(core_axis_name="c", subcore_axis_name="s")

  def body(*refs):
    if gather:
      q_hbm, src_hbm, dst_hbm, z_hbm, out_hbm, qv, srcv, dstv, accv = refs
    else:
      dst_hbm, z_hbm, out_hbm, dstv, accv = refs
    c = lax.axis_index("c")
    s = lax.axis_index("s")
    wid = c * NUM_SUBCORES + s
    base = wid * EDGES_PER_TILE

    pltpu.sync_copy(dst_hbm.at[pl.ds(base, EDGES_PER_TILE)], dstv)
    pltpu.sync_copy(z_hbm, accv)
    if gather:
      pltpu.sync_copy(src_hbm.at[pl.ds(base, EDGES_PER_TILE)], srcv)
      pltpu.sync_copy(q_hbm, qv)

    ones = jnp.ones((16,), jnp.float32)

    @pl.loop(0, EDGES_PER_TILE // 16)
    def _edges(k):
      dv = dstv[pl.ds(k * 16, 16)]
      if gather:
        sv = srcv[pl.ds(k * 16, 16)]
        sv4 = sv * NUM_CLASSES
        dv4 = dv * NUM_CLASSES
        for col in range(NUM_CLASSES):
          vals = plsc.load_gather(qv, [sv4 + col])
          plsc.addupdate_scatter(accv, [dv4 + col], vals)
      else:
        plsc.addupdate_scatter(accv, [dv, jnp.full((16,), 0, jnp.int32)],
                               ones)

    pltpu.sync_copy(accv, out_hbm.at[wid])

  if gather:
    scratch = [
        pltpu.VMEM((N_NODES * NUM_CLASSES,), jnp.float32),
        pltpu.VMEM((EDGES_PER_TILE,), jnp.int32),
        pltpu.VMEM((EDGES_PER_TILE,), jnp.int32),
        pltpu.VMEM((N_NODES * NUM_CLASSES,), jnp.float32),
    ]
    out_t = jax.ShapeDtypeStruct((NUM_TILES, N_NODES * NUM_CLASSES),
                                 jnp.float32)
  else:
    scratch = [
        pltpu.VMEM((EDGES_PER_TILE,), jnp.int32),
        pltpu.VMEM((N_NODES, 1), jnp.float32),
    ]
    out_t = jax.ShapeDtypeStruct((NUM_TILES, N_NODES, 1), jnp.float32)
  k = pl.kernel(
      body,
      out_type=out_t,
      mesh=mesh,
      scratch_types=scratch,
      compiler_params=pltpu.CompilerParams(use_tc_tiling_on_sc=False,
                                           needs_layout_passes=False),
  )
  if gather:
    return k(q4, srcf, dstf, zeros)
  return k(dstf, zeros)


def _tc_embed(xf, degp, W_pre, b_pre, W1):
  def body(degp_ref, x_ref, wp_ref, bp_ref, w1_ref, q1_ref, dinv_ref):
    deg = jnp.sum(degp_ref[...], axis=0)[:, 0] + 1.0
    dinv = lax.rsqrt(deg)
    h = jnp.maximum(
        jnp.dot(x_ref[...], wp_ref[...], preferred_element_type=jnp.float32)
        + bp_ref[...], 0.0)
    p = jnp.dot(h, w1_ref[...], preferred_element_type=jnp.float32)
    q1_ref[...] = p * dinv[:, None]
    dinv_ref[...] = dinv[:, None]

  return pl.pallas_call(
      body,
      grid=(GRID,),
      in_specs=[
          pl.BlockSpec((NUM_TILES, ROW_BLOCK, 1), lambda i: (0, i, 0)),
          pl.BlockSpec((ROW_BLOCK, 4096), lambda i: (i, 0)),
          pl.BlockSpec((4096, HIDDEN), lambda i: (0, 0)),
          pl.BlockSpec((1, HIDDEN), lambda i: (0, 0)),
          pl.BlockSpec((HIDDEN, HIDDEN), lambda i: (0, 0)),
      ],
      out_specs=[
          pl.BlockSpec((ROW_BLOCK, HIDDEN), lambda i: (i, 0)),
          pl.BlockSpec((ROW_BLOCK, 1), lambda i: (i, 0)),
      ],
      out_shape=[
          jax.ShapeDtypeStruct((N_NODES, HIDDEN), jnp.float32),
          jax.ShapeDtypeStruct((N_NODES, 1), jnp.float32),
      ],
      compiler_params=pltpu.CompilerParams(
          dimension_semantics=("arbitrary",)),
  )(degp, xf, W_pre, b_pre, W1)


def _tc_mid(aggp1, q1, dinv, W2, b1):
  def body(a0_ref, a1_ref, q1_ref, dinv_ref, w2_ref, b1_ref, q2_ref):
    s = a0_ref[...] + a1_ref[...] + q1_ref[...]
    dinv = dinv_ref[...]
    h1 = jnp.maximum(s * dinv + b1_ref[...], 0.0)
    q2_ref[...] = jnp.dot(
        h1, w2_ref[...], preferred_element_type=jnp.float32) * dinv

  return pl.pallas_call(
      body,
      grid=(GRID,),
      in_specs=[
          pl.BlockSpec((ROW_BLOCK, HIDDEN), lambda i: (i, 0)),
          pl.BlockSpec((ROW_BLOCK, HIDDEN), lambda i: (i + PART_BLOCKS, 0)),
          pl.BlockSpec((ROW_BLOCK, HIDDEN), lambda i: (i, 0)),
          pl.BlockSpec((ROW_BLOCK, 1), lambda i: (i, 0)),
          pl.BlockSpec((HIDDEN, NUM_CLASSES), lambda i: (0, 0)),
          pl.BlockSpec((1, HIDDEN), lambda i: (0, 0)),
      ],
      out_specs=pl.BlockSpec((ROW_BLOCK, NUM_CLASSES), lambda i: (i, 0)),
      out_shape=jax.ShapeDtypeStruct((N_NODES, NUM_CLASSES), jnp.float32),
      compiler_params=pltpu.CompilerParams(
          dimension_semantics=("arbitrary",)),
  )(aggp1, aggp1, q1, dinv, W2, b1)


def _tc_final(aggp2, q2, dinv, b2):
  def body(ap_ref, q2_ref, dinv_ref, b2_ref, out_ref):
    s = jnp.sum(ap_ref[...], axis=0) + q2_ref[...]
    out_ref[...] = s * dinv_ref[...] + b2_ref[...]

  return pl.pallas_call(
      body,
      grid=(GRID,),
      in_specs=[
          pl.BlockSpec((NUM_TILES, ROW_BLOCK, NUM_CLASSES),
                       lambda i: (0, i, 0)),
          pl.BlockSpec((ROW_BLOCK, NUM_CLASSES), lambda i: (i, 0)),
          pl.BlockSpec((ROW_BLOCK, 1), lambda i: (i, 0)),
          pl.BlockSpec((1, NUM_CLASSES), lambda i: (0, 0)),
      ],
      out_specs=pl.BlockSpec((ROW_BLOCK, NUM_CLASSES), lambda i: (i, 0)),
      out_shape=jax.ShapeDtypeStruct((N_NODES, NUM_CLASSES), jnp.float32),
      compiler_params=pltpu.CompilerParams(
          dimension_semantics=("arbitrary",)),
  )(aggp2, q2, dinv, b2)


def kernel(x, edge_index, W_pre, b_pre, W1, b1, W2, b2):
  n = x.shape[0]
  xf = x.astype(jnp.bfloat16).reshape(n, -1)
  srcf = edge_index[0].astype(jnp.int32)
  dstf = edge_index[1].astype(jnp.int32)
  src = srcf.reshape(N_EDGES // CHUNK, CHUNK)
  dst = dstf.reshape(N_EDGES // CHUNK, CHUNK)

  zeros128 = jnp.zeros((ACC_ROWS, HIDDEN), jnp.float32)
  zeros4 = jnp.zeros((N_NODES, NUM_CLASSES), jnp.float32)
  zeros1 = jnp.zeros((N_NODES, 1), jnp.float32)
  b_pre2 = b_pre.reshape(1, HIDDEN)
  b1_2 = b1.reshape(1, HIDDEN)
  b2_2 = b2.reshape(1, NUM_CLASSES)

  degp = _sc_flat_aggregate(None, None, dstf, zeros1, gather=False)
  q1, dinv = _tc_embed(xf, degp, W_pre.astype(jnp.bfloat16), b_pre2, W1)
  aggp1 = _sc_aggregate(q1, src, dst, zeros128, HIDDEN, gather=True)
  q2 = _tc_mid(aggp1, q1, dinv, W2, b1_2)
  aggp2 = _sc_flat_aggregate(q2.reshape(-1), srcf, dstf,
                             zeros4.reshape(-1), gather=True)
  return _tc_final(aggp2.reshape(NUM_TILES, N_NODES, NUM_CLASSES),
                   q2, dinv, b2_2)


# speedup vs baseline: 1.4621x; 1.4621x over previous
import jax
import jax.numpy as jnp
from jax import lax
from jax.experimental import pallas as pl
from jax.experimental.pallas import tpu as pltpu
from jax.experimental.pallas import tpu_sc as plsc

N_NODES = 10000
HIDDEN = 128
NUM_CLASSES = 4
N_EDGES = 320000

NUM_CORES = 2
NUM_SUBCORES = 16
NUM_TILES = NUM_CORES * NUM_SUBCORES
EDGES_PER_TILE = N_EDGES // NUM_TILES
CHUNK = 80
CHUNKS_PER_TILE = EDGES_PER_TILE // CHUNK
IDX_GROUPS = 5
GCHUNKS = CHUNKS_PER_TILE // IDX_GROUPS
NBUF = 3
ACC_ROWS = N_NODES
ZERO_SUBCORES = 10
ZERO_ROWS = ACC_ROWS // ZERO_SUBCORES

ROW_BLOCK = 400
GRID = N_NODES // ROW_BLOCK
PART_BLOCKS = ACC_ROWS // ROW_BLOCK


def _sc_aggregate(q, src, dst, zeros, d, gather):
  mesh = plsc.VectorSubcoreMesh(core_axis_name="c", subcore_axis_name="s")

  def body(*refs):
    if gather:
      (q_hbm, srcr_hbm, dstr_hbm, z_hbm, out_hbm,
       srcs_v, dsts_v, r0b, r1b, r2b, acc,
       sg0, sg1, sg2, ss0, ss1, ss2) = refs
      rows = (r0b, r1b, r2b)
      semg = (sg0, sg1, sg2)
      sems = (ss0, ss1, ss2)
    else:
      dstr_hbm, z_hbm, out_hbm, dsts_v, r0b, acc, ss0 = refs
      rows = (r0b,)
      sems = (ss0,)
    c = lax.axis_index("c")
    s = lax.axis_index("s")
    wid = c * NUM_SUBCORES + s

    r0 = s * ZERO_ROWS

    @pl.when(s < ZERO_SUBCORES)
    def _zero():
      pltpu.sync_copy(z_hbm.at[pl.ds(r0, ZERO_ROWS)],
                      acc.at[pl.ds(r0, ZERO_ROWS)])

    if not gather:
      @pl.loop(0, CHUNK)
      def _fill(r):
        for j in range(d // 16):
          rows[0][r, pl.ds(j * 16, 16)] = jnp.ones((16,), jnp.float32)

    plsc.subcore_barrier()

    t0 = wid * CHUNKS_PER_TILE

    if gather:
      @pl.loop(0, IDX_GROUPS)
      def _group(g):
        pltpu.sync_copy(dstr_hbm.at[pl.ds(t0 + g * GCHUNKS, GCHUNKS)], dsts_v)
        pltpu.sync_copy(srcr_hbm.at[pl.ds(t0 + g * GCHUNKS, GCHUNKS)], srcs_v)
        pltpu.async_copy(q_hbm.at[srcs_v.at[0]], rows[0], semg[0])
        pltpu.async_copy(q_hbm.at[srcs_v.at[1]], rows[1], semg[1])

        @pl.loop(0, GCHUNKS // NBUF)
        def _ring(tt):
          for b in range(NBUF):
            j = tt * NBUF + b
            p = (b + 2) % NBUF
            pltpu.make_async_copy(q_hbm.at[srcs_v.at[j]], rows[b],
                                  semg[b]).wait()
            pltpu.async_copy(rows[b], acc.at[dsts_v.at[j]], sems[b], add=True)

            @pl.when(jnp.logical_and(j + 2 <= GCHUNKS - 1, j >= 1))
            def _reuse():
              pltpu.make_async_copy(rows[p], acc.at[dsts_v.at[j - 1]],
                                    sems[p]).wait()
              pltpu.async_copy(q_hbm.at[srcs_v.at[j + 2]], rows[p], semg[p])

            @pl.when(jnp.logical_and(j + 2 <= GCHUNKS - 1, j < 1))
            def _first():
              pltpu.async_copy(q_hbm.at[srcs_v.at[j + 2]], rows[p], semg[p])

        last = GCHUNKS - 1
        pltpu.make_async_copy(q_hbm.at[srcs_v.at[last]], rows[0],
                              semg[0]).wait()
        pltpu.async_copy(rows[0], acc.at[dsts_v.at[last]], sems[0], add=True)
        for b in (1, 2, 0):
          j = GCHUNKS - 3 + ((b - 1) % NBUF)
          pltpu.make_async_copy(rows[b], acc.at[dsts_v.at[j]], sems[b]).wait()
    else:
      @pl.loop(0, IDX_GROUPS)
      def _dgroup(g):
        pltpu.sync_copy(dstr_hbm.at[pl.ds(t0 + g * GCHUNKS, GCHUNKS)], dsts_v)

        @pl.loop(0, GCHUNKS, step=5)
        def _deg_batch(j):
          for b in range(5):
            pltpu.async_copy(rows[0], acc.at[dsts_v.at[j + b]], sems[0],
                             add=True)
          for b in range(5):
            pltpu.make_async_copy(rows[0], acc.at[dsts_v.at[j + b]],
                                  sems[0]).wait()

    plsc.subcore_barrier()

    @pl.when(s < ZERO_SUBCORES)
    def _writeback():
      pltpu.sync_copy(acc.at[pl.ds(r0, ZERO_ROWS)],
                      out_hbm.at[pl.ds(c * ACC_ROWS + r0, ZERO_ROWS)])

  if gather:
    scratch = [
        pltpu.VMEM((GCHUNKS, CHUNK), jnp.int32),
        pltpu.VMEM((GCHUNKS, CHUNK), jnp.int32),
        pltpu.VMEM((CHUNK, d), jnp.float32),
        pltpu.VMEM((CHUNK, d), jnp.float32),
        pltpu.VMEM((CHUNK, d), jnp.float32),
        pltpu.VMEM_SHARED((ACC_ROWS, d), jnp.float32),
    ] + [pltpu.SemaphoreType.DMA] * 6
  else:
    scratch = [
        pltpu.VMEM((GCHUNKS, CHUNK), jnp.int32),
        pltpu.VMEM((CHUNK, d), jnp.float32),
        pltpu.VMEM_SHARED((ACC_ROWS, d), jnp.float32),
        pltpu.SemaphoreType.DMA,
    ]
  k = pl.kernel(
      body,
      out_type=jax.ShapeDtypeStruct((NUM_CORES * ACC_ROWS, d), jnp.float32),
      mesh=mesh,
      scratch_types=scratch,
      compiler_params=pltpu.CompilerParams(use_tc_tiling_on_sc=False),
  )
  if gather:
    return k(q, src, dst, zeros)
  return k(dst, zeros)


def _tc_embed(xf, degp, W_pre, b_pre, W1):
  def body(deg0_ref, deg1_ref, x_ref, wp_ref, bp_ref, w1_ref, q1_ref, dinv_ref):
    deg = deg0_ref[:, 0] + deg1_ref[:, 0] + 1.0
    dinv = lax.rsqrt(deg)
    h = jnp.maximum(
        jnp.dot(x_ref[...], wp_ref[...], preferred_element_type=jnp.float32)
        + bp_ref[...], 0.0)
    p = jnp.dot(h, w1_ref[...], preferred_element_type=jnp.float32)
    q1_ref[...] = p * dinv[:, None]
    dinv_ref[...] = dinv[:, None]

  return pl.pallas_call(
      body,
      grid=(GRID,),
      in_specs=[
          pl.BlockSpec((ROW_BLOCK, 16), lambda i: (i, 0)),
          pl.BlockSpec((ROW_BLOCK, 16), lambda i: (i + PART_BLOCKS, 0)),
          pl.BlockSpec((ROW_BLOCK, 4096), lambda i: (i, 0)),
          pl.BlockSpec((4096, HIDDEN), lambda i: (0, 0)),
          pl.BlockSpec((1, HIDDEN), lambda i: (0, 0)),
          pl.BlockSpec((HIDDEN, HIDDEN), lambda i: (0, 0)),
      ],
      out_specs=[
          pl.BlockSpec((ROW_BLOCK, HIDDEN), lambda i: (i, 0)),
          pl.BlockSpec((ROW_BLOCK, 1), lambda i: (i, 0)),
      ],
      out_shape=[
          jax.ShapeDtypeStruct((N_NODES, HIDDEN), jnp.float32),
          jax.ShapeDtypeStruct((N_NODES, 1), jnp.float32),
      ],
      compiler_params=pltpu.CompilerParams(
          dimension_semantics=("arbitrary",)),
  )(degp, degp, xf, W_pre, b_pre, W1)


def _tc_mid(aggp1, q1, dinv, W2p, b1):
  def body(a0_ref, a1_ref, q1_ref, dinv_ref, w2_ref, b1_ref, q2_ref):
    s = a0_ref[...] + a1_ref[...] + q1_ref[...]
    dinv = dinv_ref[...]
    h1 = jnp.maximum(s * dinv + b1_ref[...], 0.0)
    q2_ref[...] = jnp.dot(
        h1, w2_ref[...], preferred_element_type=jnp.float32) * dinv

  return pl.pallas_call(
      body,
      grid=(GRID,),
      in_specs=[
          pl.BlockSpec((ROW_BLOCK, HIDDEN), lambda i: (i, 0)),
          pl.BlockSpec((ROW_BLOCK, HIDDEN), lambda i: (i + PART_BLOCKS, 0)),
          pl.BlockSpec((ROW_BLOCK, HIDDEN), lambda i: (i, 0)),
          pl.BlockSpec((ROW_BLOCK, 1), lambda i: (i, 0)),
          pl.BlockSpec((HIDDEN, 8), lambda i: (0, 0)),
          pl.BlockSpec((1, HIDDEN), lambda i: (0, 0)),
      ],
      out_specs=pl.BlockSpec((ROW_BLOCK, 8), lambda i: (i, 0)),
      out_shape=jax.ShapeDtypeStruct((N_NODES, 8), jnp.float32),
      compiler_params=pltpu.CompilerParams(
          dimension_semantics=("arbitrary",)),
  )(aggp1, aggp1, q1, dinv, W2p, b1)


def _tc_final(aggp2, q2, dinv, b2):
  def body(a0_ref, a1_ref, q2_ref, dinv_ref, b2_ref, out_ref):
    s = a0_ref[...] + a1_ref[...] + q2_ref[...]
    out_ref[...] = s[:, :NUM_CLASSES] * dinv_ref[...] + b2_ref[...]

  return pl.pallas_call(
      body,
      grid=(GRID,),
      in_specs=[
          pl.BlockSpec((ROW_BLOCK, 8), lambda i: (i, 0)),
          pl.BlockSpec((ROW_BLOCK, 8), lambda i: (i + PART_BLOCKS, 0)),
          pl.BlockSpec((ROW_BLOCK, 8), lambda i: (i, 0)),
          pl.BlockSpec((ROW_BLOCK, 1), lambda i: (i, 0)),
          pl.BlockSpec((1, NUM_CLASSES), lambda i: (0, 0)),
      ],
      out_specs=pl.BlockSpec((ROW_BLOCK, NUM_CLASSES), lambda i: (i, 0)),
      out_shape=jax.ShapeDtypeStruct((N_NODES, NUM_CLASSES), jnp.float32),
      compiler_params=pltpu.CompilerParams(
          dimension_semantics=("arbitrary",)),
  )(aggp2, aggp2, q2, dinv, b2)


def kernel(x, edge_index, W_pre, b_pre, W1, b1, W2, b2):
  n = x.shape[0]
  xf = x.astype(jnp.bfloat16).reshape(n, -1)
  src = edge_index[0].astype(jnp.int32).reshape(N_EDGES // CHUNK, CHUNK)
  dst = edge_index[1].astype(jnp.int32).reshape(N_EDGES // CHUNK, CHUNK)

  zeros16 = jnp.zeros((ACC_ROWS, 16), jnp.float32)
  zeros8 = jnp.zeros((ACC_ROWS, 8), jnp.float32)
  zeros128 = jnp.zeros((ACC_ROWS, HIDDEN), jnp.float32)
  W2p = jnp.pad(W2, ((0, 0), (0, 8 - NUM_CLASSES)))
  b_pre2 = b_pre.reshape(1, HIDDEN)
  b1_2 = b1.reshape(1, HIDDEN)
  b2_2 = b2.reshape(1, NUM_CLASSES)

  degp = _sc_aggregate(None, None, dst, zeros16, 16, gather=False)
  q1, dinv = _tc_embed(xf, degp, W_pre.astype(jnp.bfloat16), b_pre2, W1)
  aggp1 = _sc_aggregate(q1, src, dst, zeros128, HIDDEN, gather=True)
  q2 = _tc_mid(aggp1, q1, dinv, W2p, b1_2)
  aggp2 = _sc_aggregate(q2, src, dst, zeros8, 8, gather=True)
  return _tc_final(aggp2, q2, dinv, b2_2)


# speedup vs baseline: 1.4634x; 1.0009x over previous
import jax
import jax.numpy as jnp
from jax import lax
from jax.experimental import pallas as pl
from jax.experimental.pallas import tpu as pltpu
from jax.experimental.pallas import tpu_sc as plsc

N_NODES = 10000
HIDDEN = 128
NUM_CLASSES = 4
N_EDGES = 320000

NUM_CORES = 2
NUM_SUBCORES = 16
NUM_TILES = NUM_CORES * NUM_SUBCORES
EDGES_PER_TILE = N_EDGES // NUM_TILES
CHUNK = 80
CHUNKS_PER_TILE = EDGES_PER_TILE // CHUNK
IDX_GROUPS = 5
GCHUNKS = CHUNKS_PER_TILE // IDX_GROUPS
NBUF = 3
ACC_ROWS = N_NODES
ZERO_SUBCORES = 10
ZERO_ROWS = ACC_ROWS // ZERO_SUBCORES

ROW_BLOCK = 400
GRID = N_NODES // ROW_BLOCK
PART_BLOCKS = ACC_ROWS // ROW_BLOCK


def _sc_aggregate(q, src, dst, zeros, d, gather):
  mesh = plsc.VectorSubcoreMesh(core_axis_name="c", subcore_axis_name="s")

  def body(*refs):
    if gather:
      (q_hbm, srcr_hbm, dstr_hbm, z_hbm, out_hbm,
       srcs_v, dsts_v, r0b, r1b, r2b, acc,
       sg0, sg1, sg2, ss0, ss1, ss2) = refs
      rows = (r0b, r1b, r2b)
      semg = (sg0, sg1, sg2)
      sems = (ss0, ss1, ss2)
    else:
      dstr_hbm, z_hbm, out_hbm, dsts_v, r0b, acc, ss0 = refs
      rows = (r0b,)
      sems = (ss0,)
    c = lax.axis_index("c")
    s = lax.axis_index("s")
    wid = c * NUM_SUBCORES + s

    r0 = s * ZERO_ROWS

    @pl.when(s < ZERO_SUBCORES)
    def _zero():
      pltpu.sync_copy(z_hbm.at[pl.ds(r0, ZERO_ROWS)],
                      acc.at[pl.ds(r0, ZERO_ROWS)])

    if not gather:
      @pl.loop(0, CHUNK)
      def _fill(r):
        for j in range(d // 16):
          rows[0][r, pl.ds(j * 16, 16)] = jnp.ones((16,), jnp.float32)

    plsc.subcore_barrier()

    t0 = wid * CHUNKS_PER_TILE

    if gather:
      @pl.loop(0, IDX_GROUPS)
      def _group(g):
        pltpu.sync_copy(dstr_hbm.at[pl.ds(t0 + g * GCHUNKS, GCHUNKS)], dsts_v)
        pltpu.sync_copy(srcr_hbm.at[pl.ds(t0 + g * GCHUNKS, GCHUNKS)], srcs_v)
        pltpu.async_copy(q_hbm.at[srcs_v.at[0]], rows[0], semg[0])
        pltpu.async_copy(q_hbm.at[srcs_v.at[1]], rows[1], semg[1])

        @pl.loop(0, GCHUNKS // NBUF)
        def _ring(tt):
          for b in range(NBUF):
            j = tt * NBUF + b
            p = (b + 2) % NBUF
            pltpu.make_async_copy(q_hbm.at[srcs_v.at[j]], rows[b],
                                  semg[b]).wait()
            pltpu.async_copy(rows[b], acc.at[dsts_v.at[j]], sems[b], add=True)

            @pl.when(jnp.logical_and(j + 2 <= GCHUNKS - 1, j >= 1))
            def _reuse():
              pltpu.make_async_copy(rows[p], acc.at[dsts_v.at[j - 1]],
                                    sems[p]).wait()
              pltpu.async_copy(q_hbm.at[srcs_v.at[j + 2]], rows[p], semg[p])

            @pl.when(jnp.logical_and(j + 2 <= GCHUNKS - 1, j < 1))
            def _first():
              pltpu.async_copy(q_hbm.at[srcs_v.at[j + 2]], rows[p], semg[p])

        last = GCHUNKS - 1
        pltpu.make_async_copy(q_hbm.at[srcs_v.at[last]], rows[0],
                              semg[0]).wait()
        pltpu.async_copy(rows[0], acc.at[dsts_v.at[last]], sems[0], add=True)
        for b in (1, 2, 0):
          j = GCHUNKS - 3 + ((b - 1) % NBUF)
          pltpu.make_async_copy(rows[b], acc.at[dsts_v.at[j]], sems[b]).wait()
    else:
      @pl.loop(0, IDX_GROUPS)
      def _dgroup(g):
        pltpu.sync_copy(dstr_hbm.at[pl.ds(t0 + g * GCHUNKS, GCHUNKS)], dsts_v)

        @pl.loop(0, GCHUNKS, step=5)
        def _deg_batch(j):
          for b in range(5):
            pltpu.async_copy(rows[0], acc.at[dsts_v.at[j + b]], sems[0],
                             add=True)
          for b in range(5):
            pltpu.make_async_copy(rows[0], acc.at[dsts_v.at[j + b]],
                                  sems[0]).wait()

    plsc.subcore_barrier()

    @pl.when(s < ZERO_SUBCORES)
    def _writeback():
      pltpu.sync_copy(acc.at[pl.ds(r0, ZERO_ROWS)],
                      out_hbm.at[pl.ds(c * ACC_ROWS + r0, ZERO_ROWS)])

  if gather:
    scratch = [
        pltpu.VMEM((GCHUNKS, CHUNK), jnp.int32),
        pltpu.VMEM((GCHUNKS, CHUNK), jnp.int32),
        pltpu.VMEM((CHUNK, d), jnp.float32),
        pltpu.VMEM((CHUNK, d), jnp.float32),
        pltpu.VMEM((CHUNK, d), jnp.float32),
        pltpu.VMEM_SHARED((ACC_ROWS, d), jnp.float32),
    ] + [pltpu.SemaphoreType.DMA] * 6
  else:
    scratch = [
        pltpu.VMEM((GCHUNKS, CHUNK), jnp.int32),
        pltpu.VMEM((CHUNK, d), jnp.float32),
        pltpu.VMEM_SHARED((ACC_ROWS, d), jnp.float32),
        pltpu.SemaphoreType.DMA,
    ]
  k = pl.kernel(
      body,
      out_type=jax.ShapeDtypeStruct((NUM_CORES * ACC_ROWS, d), jnp.float32),
      mesh=mesh,
      scratch_types=scratch,
      compiler_params=pltpu.CompilerParams(use_tc_tiling_on_sc=False),
  )
  if gather:
    return k(q, src, dst, zeros)
  return k(dst, zeros)


def _tc_embed(xf, degp, W_pre, b_pre, W1):
  def body(deg0_ref, deg1_ref, x_ref, wp_ref, bp_ref, w1_ref, q1_ref, dinv_ref):
    deg = deg0_ref[:, 0] + deg1_ref[:, 0] + 1.0
    dinv = lax.rsqrt(deg)
    h = jnp.maximum(
        jnp.dot(x_ref[...], wp_ref[...], preferred_element_type=jnp.float32)
        + bp_ref[...], 0.0)
    p = jnp.dot(h, w1_ref[...], preferred_element_type=jnp.float32)
    q1_ref[...] = p * dinv[:, None]
    dinv_ref[...] = dinv[:, None]

  return pl.pallas_call(
      body,
      grid=(GRID,),
      in_specs=[
          pl.BlockSpec((ROW_BLOCK, 16), lambda i: (i, 0)),
          pl.BlockSpec((ROW_BLOCK, 16), lambda i: (i + PART_BLOCKS, 0)),
          pl.BlockSpec((ROW_BLOCK, 4096), lambda i: (i, 0)),
          pl.BlockSpec((4096, HIDDEN), lambda i: (0, 0)),
          pl.BlockSpec((1, HIDDEN), lambda i: (0, 0)),
          pl.BlockSpec((HIDDEN, HIDDEN), lambda i: (0, 0)),
      ],
      out_specs=[
          pl.BlockSpec((ROW_BLOCK, HIDDEN), lambda i: (i, 0)),
          pl.BlockSpec((ROW_BLOCK, 1), lambda i: (i, 0)),
      ],
      out_shape=[
          jax.ShapeDtypeStruct((N_NODES, HIDDEN), jnp.float32),
          jax.ShapeDtypeStruct((N_NODES, 1), jnp.float32),
      ],
      compiler_params=pltpu.CompilerParams(
          dimension_semantics=("arbitrary",),
          allow_input_fusion=[False, False, True, False, False, False]),
  )(degp, degp, xf, W_pre, b_pre, W1)


def _tc_mid(aggp1, q1, dinv, W2p, b1):
  def body(a0_ref, a1_ref, q1_ref, dinv_ref, w2_ref, b1_ref, q2_ref):
    s = a0_ref[...] + a1_ref[...] + q1_ref[...]
    dinv = dinv_ref[...]
    h1 = jnp.maximum(s * dinv + b1_ref[...], 0.0)
    q2_ref[...] = jnp.dot(
        h1, w2_ref[...], preferred_element_type=jnp.float32) * dinv

  return pl.pallas_call(
      body,
      grid=(GRID,),
      in_specs=[
          pl.BlockSpec((ROW_BLOCK, HIDDEN), lambda i: (i, 0)),
          pl.BlockSpec((ROW_BLOCK, HIDDEN), lambda i: (i + PART_BLOCKS, 0)),
          pl.BlockSpec((ROW_BLOCK, HIDDEN), lambda i: (i, 0)),
          pl.BlockSpec((ROW_BLOCK, 1), lambda i: (i, 0)),
          pl.BlockSpec((HIDDEN, 8), lambda i: (0, 0)),
          pl.BlockSpec((1, HIDDEN), lambda i: (0, 0)),
      ],
      out_specs=pl.BlockSpec((ROW_BLOCK, 8), lambda i: (i, 0)),
      out_shape=jax.ShapeDtypeStruct((N_NODES, 8), jnp.float32),
      compiler_params=pltpu.CompilerParams(
          dimension_semantics=("arbitrary",)),
  )(aggp1, aggp1, q1, dinv, W2p, b1)


def _tc_final(aggp2, q2, dinv, b2):
  def body(a0_ref, a1_ref, q2_ref, dinv_ref, b2_ref, out_ref):
    s = a0_ref[...] + a1_ref[...] + q2_ref[...]
    out_ref[...] = s[:, :NUM_CLASSES] * dinv_ref[...] + b2_ref[...]

  return pl.pallas_call(
      body,
      grid=(GRID,),
      in_specs=[
          pl.BlockSpec((ROW_BLOCK, 8), lambda i: (i, 0)),
          pl.BlockSpec((ROW_BLOCK, 8), lambda i: (i + PART_BLOCKS, 0)),
          pl.BlockSpec((ROW_BLOCK, 8), lambda i: (i, 0)),
          pl.BlockSpec((ROW_BLOCK, 1), lambda i: (i, 0)),
          pl.BlockSpec((1, NUM_CLASSES), lambda i: (0, 0)),
      ],
      out_specs=pl.BlockSpec((ROW_BLOCK, NUM_CLASSES), lambda i: (i, 0)),
      out_shape=jax.ShapeDtypeStruct((N_NODES, NUM_CLASSES), jnp.float32),
      compiler_params=pltpu.CompilerParams(
          dimension_semantics=("arbitrary",)),
  )(aggp2, aggp2, q2, dinv, b2)


def kernel(x, edge_index, W_pre, b_pre, W1, b1, W2, b2):
  n = x.shape[0]
  xf = x.astype(jnp.bfloat16).reshape(n, -1)
  src = edge_index[0].astype(jnp.int32).reshape(N_EDGES // CHUNK, CHUNK)
  dst = edge_index[1].astype(jnp.int32).reshape(N_EDGES // CHUNK, CHUNK)

  zeros16 = jnp.zeros((ACC_ROWS, 16), jnp.float32)
  zeros8 = jnp.zeros((ACC_ROWS, 8), jnp.float32)
  zeros128 = jnp.zeros((ACC_ROWS, HIDDEN), jnp.float32)
  W2p = jnp.pad(W2, ((0, 0), (0, 8 - NUM_CLASSES)))
  b_pre2 = b_pre.reshape(1, HIDDEN)
  b1_2 = b1.reshape(1, HIDDEN)
  b2_2 = b2.reshape(1, NUM_CLASSES)

  degp = _sc_aggregate(None, None, dst, zeros16, 16, gather=False)
  q1, dinv = _tc_embed(xf, degp, W_pre.astype(jnp.bfloat16), b_pre2, W1)
  aggp1 = _sc_aggregate(q1, src, dst, zeros128, HIDDEN, gather=True)
  q2 = _tc_mid(aggp1, q1, dinv, W2p, b1_2)
  aggp2 = _sc_aggregate(q2, src, dst, zeros8, 8, gather=True)
  return _tc_final(aggp2, q2, dinv, b2_2)


# speedup vs baseline: 1.4840x; 1.0141x over previous
import jax
import jax.numpy as jnp
from jax import lax
from jax.experimental import pallas as pl
from jax.experimental.pallas import tpu as pltpu
from jax.experimental.pallas import tpu_sc as plsc

N_NODES = 10000
HIDDEN = 128
NUM_CLASSES = 4
N_EDGES = 320000

NUM_CORES = 2
NUM_SUBCORES = 16
NUM_TILES = NUM_CORES * NUM_SUBCORES
EDGES_PER_TILE = N_EDGES // NUM_TILES
CHUNK = 80
CHUNKS_PER_TILE = EDGES_PER_TILE // CHUNK
IDX_GROUPS = 5
GCHUNKS = CHUNKS_PER_TILE // IDX_GROUPS
NBUF = 4
ACC_ROWS = N_NODES
ZERO_SUBCORES = 10
ZERO_ROWS = ACC_ROWS // ZERO_SUBCORES

ROW_BLOCK = 400
GRID = N_NODES // ROW_BLOCK
PART_BLOCKS = ACC_ROWS // ROW_BLOCK


def _sc_aggregate(q, src, dst, zeros, d, gather):
  mesh = plsc.VectorSubcoreMesh(core_axis_name="c", subcore_axis_name="s")

  def body(*refs):
    if gather:
      (q_hbm, srcr_hbm, dstr_hbm, z_hbm, out_hbm,
       srcs_v, dsts_v, r0b, r1b, r2b, r3b, acc,
       sg0, sg1, sg2, sg3, ss0, ss1, ss2, ss3) = refs
      rows = (r0b, r1b, r2b, r3b)
      semg = (sg0, sg1, sg2, sg3)
      sems = (ss0, ss1, ss2, ss3)
    else:
      dstr_hbm, z_hbm, out_hbm, dsts_v, r0b, acc, ss0 = refs
      rows = (r0b,)
      sems = (ss0,)
    c = lax.axis_index("c")
    s = lax.axis_index("s")
    wid = c * NUM_SUBCORES + s

    r0 = s * ZERO_ROWS

    @pl.when(s < ZERO_SUBCORES)
    def _zero():
      pltpu.sync_copy(z_hbm.at[pl.ds(r0, ZERO_ROWS)],
                      acc.at[pl.ds(r0, ZERO_ROWS)])

    if not gather:
      @pl.loop(0, CHUNK)
      def _fill(r):
        for j in range(d // 16):
          rows[0][r, pl.ds(j * 16, 16)] = jnp.ones((16,), jnp.float32)

    plsc.subcore_barrier()

    t0 = wid * CHUNKS_PER_TILE

    if gather:
      @pl.loop(0, IDX_GROUPS)
      def _group(g):
        pltpu.sync_copy(dstr_hbm.at[pl.ds(t0 + g * GCHUNKS, GCHUNKS)], dsts_v)
        pltpu.sync_copy(srcr_hbm.at[pl.ds(t0 + g * GCHUNKS, GCHUNKS)], srcs_v)
        for pb in range(NBUF - 1):
          pltpu.async_copy(q_hbm.at[srcs_v.at[pb]], rows[pb], semg[pb])

        @pl.loop(0, GCHUNKS // NBUF)
        def _ring(tt):
          for b in range(NBUF):
            j = tt * NBUF + b
            p = (b + NBUF - 1) % NBUF
            pltpu.make_async_copy(q_hbm.at[srcs_v.at[j]], rows[b],
                                  semg[b]).wait()
            pltpu.async_copy(rows[b], acc.at[dsts_v.at[j]], sems[b], add=True)

            @pl.when(jnp.logical_and(j + NBUF - 1 <= GCHUNKS - 1, j >= 1))
            def _reuse():
              pltpu.make_async_copy(rows[p], acc.at[dsts_v.at[j - 1]],
                                    sems[p]).wait()
              pltpu.async_copy(q_hbm.at[srcs_v.at[j + NBUF - 1]], rows[p],
                               semg[p])

            @pl.when(jnp.logical_and(j + NBUF - 1 <= GCHUNKS - 1, j < 1))
            def _first():
              pltpu.async_copy(q_hbm.at[srcs_v.at[j + NBUF - 1]], rows[p],
                               semg[p])

        last = GCHUNKS - 1
        pltpu.make_async_copy(q_hbm.at[srcs_v.at[last]], rows[0],
                              semg[0]).wait()
        pltpu.async_copy(rows[0], acc.at[dsts_v.at[last]], sems[0], add=True)
        for b in list(range(1, NBUF)) + [0]:
          j = GCHUNKS - NBUF + ((b - 1) % NBUF)
          pltpu.make_async_copy(rows[b], acc.at[dsts_v.at[j]], sems[b]).wait()
    else:
      @pl.loop(0, IDX_GROUPS)
      def _dgroup(g):
        pltpu.sync_copy(dstr_hbm.at[pl.ds(t0 + g * GCHUNKS, GCHUNKS)], dsts_v)

        @pl.loop(0, GCHUNKS, step=5)
        def _deg_batch(j):
          for b in range(5):
            pltpu.async_copy(rows[0], acc.at[dsts_v.at[j + b]], sems[0],
                             add=True)
          for b in range(5):
            pltpu.make_async_copy(rows[0], acc.at[dsts_v.at[j + b]],
                                  sems[0]).wait()

    plsc.subcore_barrier()

    @pl.when(s < ZERO_SUBCORES)
    def _writeback():
      pltpu.sync_copy(acc.at[pl.ds(r0, ZERO_ROWS)],
                      out_hbm.at[pl.ds(c * ACC_ROWS + r0, ZERO_ROWS)])

  if gather:
    scratch = [
        pltpu.VMEM((GCHUNKS, CHUNK), jnp.int32),
        pltpu.VMEM((GCHUNKS, CHUNK), jnp.int32),
        pltpu.VMEM((CHUNK, d), jnp.float32),
        pltpu.VMEM((CHUNK, d), jnp.float32),
        pltpu.VMEM((CHUNK, d), jnp.float32),
        pltpu.VMEM((CHUNK, d), jnp.float32),
        pltpu.VMEM_SHARED((ACC_ROWS, d), jnp.float32),
    ] + [pltpu.SemaphoreType.DMA] * 8
  else:
    scratch = [
        pltpu.VMEM((GCHUNKS, CHUNK), jnp.int32),
        pltpu.VMEM((CHUNK, d), jnp.float32),
        pltpu.VMEM_SHARED((ACC_ROWS, d), jnp.float32),
        pltpu.SemaphoreType.DMA,
    ]
  k = pl.kernel(
      body,
      out_type=jax.ShapeDtypeStruct((NUM_CORES * ACC_ROWS, d), jnp.float32),
      mesh=mesh,
      scratch_types=scratch,
      compiler_params=pltpu.CompilerParams(use_tc_tiling_on_sc=False),
  )
  if gather:
    return k(q, src, dst, zeros)
  return k(dst, zeros)


def _tc_embed(xf, degp, W_pre, b_pre, W1):
  def body(deg0_ref, deg1_ref, x_ref, wp_ref, bp_ref, w1_ref, q1_ref, dinv_ref):
    deg = deg0_ref[:, 0] + deg1_ref[:, 0] + 1.0
    dinv = lax.rsqrt(deg)
    h = jnp.maximum(
        jnp.dot(x_ref[...], wp_ref[...], preferred_element_type=jnp.float32)
        + bp_ref[...], 0.0)
    p = jnp.dot(h, w1_ref[...], preferred_element_type=jnp.float32)
    q1_ref[...] = p * dinv[:, None]
    dinv_ref[...] = dinv[:, None]

  return pl.pallas_call(
      body,
      grid=(GRID,),
      in_specs=[
          pl.BlockSpec((ROW_BLOCK, 16), lambda i: (i, 0)),
          pl.BlockSpec((ROW_BLOCK, 16), lambda i: (i + PART_BLOCKS, 0)),
          pl.BlockSpec((ROW_BLOCK, 4096), lambda i: (i, 0)),
          pl.BlockSpec((4096, HIDDEN), lambda i: (0, 0)),
          pl.BlockSpec((1, HIDDEN), lambda i: (0, 0)),
          pl.BlockSpec((HIDDEN, HIDDEN), lambda i: (0, 0)),
      ],
      out_specs=[
          pl.BlockSpec((ROW_BLOCK, HIDDEN), lambda i: (i, 0)),
          pl.BlockSpec((ROW_BLOCK, 1), lambda i: (i, 0)),
      ],
      out_shape=[
          jax.ShapeDtypeStruct((N_NODES, HIDDEN), jnp.float32),
          jax.ShapeDtypeStruct((N_NODES, 1), jnp.float32),
      ],
      compiler_params=pltpu.CompilerParams(
          dimension_semantics=("arbitrary",),
          allow_input_fusion=[False, False, True, False, False, False]),
  )(degp, degp, xf, W_pre, b_pre, W1)


def _tc_mid(aggp1, q1, dinv, W2p, b1):
  def body(a0_ref, a1_ref, q1_ref, dinv_ref, w2_ref, b1_ref, q2_ref):
    s = a0_ref[...] + a1_ref[...] + q1_ref[...]
    dinv = dinv_ref[...]
    h1 = jnp.maximum(s * dinv + b1_ref[...], 0.0)
    q2_ref[...] = jnp.dot(
        h1, w2_ref[...], preferred_element_type=jnp.float32) * dinv

  return pl.pallas_call(
      body,
      grid=(GRID,),
      in_specs=[
          pl.BlockSpec((ROW_BLOCK, HIDDEN), lambda i: (i, 0)),
          pl.BlockSpec((ROW_BLOCK, HIDDEN), lambda i: (i + PART_BLOCKS, 0)),
          pl.BlockSpec((ROW_BLOCK, HIDDEN), lambda i: (i, 0)),
          pl.BlockSpec((ROW_BLOCK, 1), lambda i: (i, 0)),
          pl.BlockSpec((HIDDEN, 8), lambda i: (0, 0)),
          pl.BlockSpec((1, HIDDEN), lambda i: (0, 0)),
      ],
      out_specs=pl.BlockSpec((ROW_BLOCK, 8), lambda i: (i, 0)),
      out_shape=jax.ShapeDtypeStruct((N_NODES, 8), jnp.float32),
      compiler_params=pltpu.CompilerParams(
          dimension_semantics=("arbitrary",)),
  )(aggp1, aggp1, q1, dinv, W2p, b1)


def _tc_final(aggp2, q2, dinv, b2):
  def body(a0_ref, a1_ref, q2_ref, dinv_ref, b2_ref, out_ref):
    s = a0_ref[...] + a1_ref[...] + q2_ref[...]
    out_ref[...] = s[:, :NUM_CLASSES] * dinv_ref[...] + b2_ref[...]

  return pl.pallas_call(
      body,
      grid=(GRID,),
      in_specs=[
          pl.BlockSpec((ROW_BLOCK, 8), lambda i: (i, 0)),
          pl.BlockSpec((ROW_BLOCK, 8), lambda i: (i + PART_BLOCKS, 0)),
          pl.BlockSpec((ROW_BLOCK, 8), lambda i: (i, 0)),
          pl.BlockSpec((ROW_BLOCK, 1), lambda i: (i, 0)),
          pl.BlockSpec((1, NUM_CLASSES), lambda i: (0, 0)),
      ],
      out_specs=pl.BlockSpec((ROW_BLOCK, NUM_CLASSES), lambda i: (i, 0)),
      out_shape=jax.ShapeDtypeStruct((N_NODES, NUM_CLASSES), jnp.float32),
      compiler_params=pltpu.CompilerParams(
          dimension_semantics=("arbitrary",)),
  )(aggp2, aggp2, q2, dinv, b2)


def kernel(x, edge_index, W_pre, b_pre, W1, b1, W2, b2):
  n = x.shape[0]
  xf = x.astype(jnp.bfloat16).reshape(n, -1)
  src = edge_index[0].astype(jnp.int32).reshape(N_EDGES // CHUNK, CHUNK)
  dst = edge_index[1].astype(jnp.int32).reshape(N_EDGES // CHUNK, CHUNK)

  zeros16 = jnp.zeros((ACC_ROWS, 16), jnp.float32)
  zeros8 = jnp.zeros((ACC_ROWS, 8), jnp.float32)
  zeros128 = jnp.zeros((ACC_ROWS, HIDDEN), jnp.float32)
  W2p = jnp.pad(W2, ((0, 0), (0, 8 - NUM_CLASSES)))
  b_pre2 = b_pre.reshape(1, HIDDEN)
  b1_2 = b1.reshape(1, HIDDEN)
  b2_2 = b2.reshape(1, NUM_CLASSES)

  degp = _sc_aggregate(None, None, dst, zeros16, 16, gather=False)
  q1, dinv = _tc_embed(xf, degp, W_pre.astype(jnp.bfloat16), b_pre2, W1)
  aggp1 = _sc_aggregate(q1, src, dst, zeros128, HIDDEN, gather=True)
  q2 = _tc_mid(aggp1, q1, dinv, W2p, b1_2)
  aggp2 = _sc_aggregate(q2, src, dst, zeros8, 8, gather=True)
  return _tc_final(aggp2, q2, dinv, b2_2)
